# baseline (device time: 104616 ns/iter reference)
import jax
import jax.numpy as jnp
from jax import lax
from jax.experimental import pallas as pl
from jax.experimental.pallas import tpu as pltpu

N_DEV = 8
RLIST = [1, 6, 2, 5, 3, 4, 7]
NSUB = 2


def kernel(x, w_mat):
    m_per, k = x.shape
    _, n = w_mat.shape
    n_per = n // N_DEV
    n_sub = n_per // NSUB
    m = m_per * N_DEV

    def body(x_ref, w_ref, out_ref, w_bufs, y_bufs, recv_bufs,
             w_sems, send_sems, recv_sems):
        my = lax.axis_index("i")

        def target(c):
            return (my ^ RLIST[c]) if c < N_DEV - 1 else my

        def w_copy(s):
            off = target(s // NSUB) * n_per + (s % NSUB) * n_sub
            return pltpu.make_async_copy(
                w_ref.at[:, pl.ds(off, n_sub)],
                w_bufs.at[s % 2],
                w_sems.at[s % 2],
            )

        w_copy(0).start()

        def drain(c):
            src = my ^ RLIST[c]
            recv = pltpu.make_async_remote_copy(
                src_ref=y_bufs.at[0],
                dst_ref=recv_bufs.at[src],
                send_sem=send_sems.at[0],
                recv_sem=recv_sems.at[src],
                device_id=(src,),
                device_id_type=pl.DeviceIdType.MESH,
            )
            recv.wait_recv()
            out_ref[pl.ds(src * m_per, m_per), :] = (
                recv_bufs[src].astype(jnp.float32))

        send_rdmas = []
        for s in range(N_DEV * NSUB):
            c, h = s // NSUB, s % NSUB
            if s + 1 < N_DEV * NSUB:
                w_copy(s + 1).start()
            w_copy(s).wait()

            y = jnp.dot(x_ref[...], w_bufs[s % 2],
                        preferred_element_type=jnp.float32)
            y = y * jax.nn.sigmoid(y)

            if c == N_DEV - 1:
                out_ref[pl.ds(my * m_per, m_per),
                        pl.ds(h * n_sub, n_sub)] = y
            else:
                y_bufs[c, :, pl.ds(h * n_sub, n_sub)] = y.astype(jnp.bfloat16)
                if h == NSUB - 1:
                    rdma = pltpu.make_async_remote_copy(
                        src_ref=y_bufs.at[c],
                        dst_ref=recv_bufs.at[my],
                        send_sem=send_sems.at[c],
                        recv_sem=recv_sems.at[my],
                        device_id=(target(c),),
                        device_id_type=pl.DeviceIdType.MESH,
                    )
                    rdma.start()
                    send_rdmas.append(rdma)

        for c in range(N_DEV - 1):
            drain(c)
        for r in send_rdmas:
            r.wait_send()

    return pl.pallas_call(
        body,
        out_shape=jax.ShapeDtypeStruct((m, n_per), jnp.float32),
        in_specs=[
            pl.BlockSpec(memory_space=pltpu.VMEM),
            pl.BlockSpec(memory_space=pltpu.MemorySpace.HBM),
        ],
        out_specs=pl.BlockSpec(memory_space=pltpu.VMEM),
        scratch_shapes=[
            pltpu.VMEM((2, k, n_per // NSUB), jnp.float32),
            pltpu.VMEM((N_DEV - 1, m_per, n_per), jnp.bfloat16),
            pltpu.VMEM((N_DEV, m_per, n_per), jnp.bfloat16),
            pltpu.SemaphoreType.DMA((2,)),
            pltpu.SemaphoreType.DMA((N_DEV - 1,)),
            pltpu.SemaphoreType.DMA((N_DEV,)),
        ],
        compiler_params=pltpu.CompilerParams(
            vmem_limit_bytes=64 * 1024 * 1024,
        ),
    )(x, w_mat)


# device time: 100367 ns/iter; 1.0423x vs baseline; 1.0423x over previous
import jax
import jax.numpy as jnp
from jax import lax
from jax.experimental import pallas as pl
from jax.experimental.pallas import tpu as pltpu

N_DEV = 8
RLIST = [1, 6, 2, 5, 3, 4, 7]
NSUB = 2


def kernel(x, w_mat):
    m_per, k = x.shape
    _, n = w_mat.shape
    n_per = n // N_DEV
    n_sub = n_per // NSUB
    m = m_per * N_DEV

    def body(x_ref, w_ref, out_ref, w_bufs, y_bufs, recv_bufs, stage,
             w_sems, send_sems, recv_sems, stage_sems):
        my = lax.axis_index("i")
        stage_copies = [None, None]

        def stage_out(slot, rows, y_f32, cols=None):
            if stage_copies[slot] is not None:
                stage_copies[slot].wait()
            if cols is None:
                stage[slot] = y_f32
                src = stage.at[slot]
                dst = out_ref.at[pl.ds(rows, m_per), :]
            else:
                stage[slot, :, pl.ds(cols, n_sub)] = y_f32
                src = stage.at[slot, :, pl.ds(cols, n_sub)]
                dst = out_ref.at[pl.ds(rows, m_per), pl.ds(cols, n_sub)]
            cp = pltpu.make_async_copy(src, dst, stage_sems.at[slot])
            cp.start()
            stage_copies[slot] = cp

        def target(c):
            return (my ^ RLIST[c]) if c < N_DEV - 1 else my

        def w_copy(s):
            off = target(s // NSUB) * n_per + (s % NSUB) * n_sub
            return pltpu.make_async_copy(
                w_ref.at[:, pl.ds(off, n_sub)],
                w_bufs.at[s % 2],
                w_sems.at[s % 2],
            )

        w_copy(0).start()

        def drain(c):
            src = my ^ RLIST[c]
            recv = pltpu.make_async_remote_copy(
                src_ref=y_bufs.at[0],
                dst_ref=recv_bufs.at[src],
                send_sem=send_sems.at[0],
                recv_sem=recv_sems.at[src],
                device_id=(src,),
                device_id_type=pl.DeviceIdType.MESH,
            )
            recv.wait_recv()
            stage_out(c % 2, src * m_per, recv_bufs[src].astype(jnp.float32))

        send_rdmas = []
        for s in range(N_DEV * NSUB):
            c, h = s // NSUB, s % NSUB
            if s + 1 < N_DEV * NSUB:
                w_copy(s + 1).start()
            w_copy(s).wait()

            y = jnp.dot(x_ref[...], w_bufs[s % 2],
                        preferred_element_type=jnp.float32)
            y = y * jax.nn.sigmoid(y)

            if c == N_DEV - 1:
                stage_out(h, my * m_per, y, cols=h * n_sub)
            else:
                y_bufs[c, :, pl.ds(h * n_sub, n_sub)] = y.astype(jnp.bfloat16)
                if h == NSUB - 1:
                    rdma = pltpu.make_async_remote_copy(
                        src_ref=y_bufs.at[c],
                        dst_ref=recv_bufs.at[my],
                        send_sem=send_sems.at[c],
                        recv_sem=recv_sems.at[my],
                        device_id=(target(c),),
                        device_id_type=pl.DeviceIdType.MESH,
                    )
                    rdma.start()
                    send_rdmas.append(rdma)

        for c in range(N_DEV - 1):
            drain(c)
        for cp in stage_copies:
            cp.wait()
        for r in send_rdmas:
            r.wait_send()

    return pl.pallas_call(
        body,
        out_shape=jax.ShapeDtypeStruct((m, n_per), jnp.float32),
        in_specs=[
            pl.BlockSpec(memory_space=pltpu.VMEM),
            pl.BlockSpec(memory_space=pltpu.MemorySpace.HBM),
        ],
        out_specs=pl.BlockSpec(memory_space=pltpu.MemorySpace.HBM),
        scratch_shapes=[
            pltpu.VMEM((2, k, n_per // NSUB), jnp.float32),
            pltpu.VMEM((N_DEV - 1, m_per, n_per), jnp.bfloat16),
            pltpu.VMEM((N_DEV, m_per, n_per), jnp.bfloat16),
            pltpu.VMEM((2, m_per, n_per), jnp.float32),
            pltpu.SemaphoreType.DMA((2,)),
            pltpu.SemaphoreType.DMA((N_DEV - 1,)),
            pltpu.SemaphoreType.DMA((N_DEV,)),
            pltpu.SemaphoreType.DMA((2,)),
        ],
        compiler_params=pltpu.CompilerParams(
            vmem_limit_bytes=64 * 1024 * 1024,
        ),
    )(x, w_mat)


# device time: 90935 ns/iter; 1.1504x vs baseline; 1.1037x over previous
import jax
import jax.numpy as jnp
from jax import lax
from jax.experimental import pallas as pl
from jax.experimental.pallas import tpu as pltpu

N_DEV = 8
RLIST = [7, 4, 6, 5, 3, 2, 1]
NSUB = 2


def kernel(x, w_mat):
    m_per, k = x.shape
    _, n = w_mat.shape
    n_per = n // N_DEV
    n_sub = n_per // NSUB
    m = m_per * N_DEV

    def body(x_ref, w_ref, out_ref, w_bufs, y_bufs, recv_bufs, stage,
             w_sems, send_sems, recv_sems, stage_sems):
        my = lax.axis_index("i")
        stage_copies = [None, None]

        def stage_out(slot, rows, y_f32, cols=None):
            if stage_copies[slot] is not None:
                stage_copies[slot].wait()
            if cols is None:
                stage[slot] = y_f32
                src = stage.at[slot]
                dst = out_ref.at[pl.ds(rows, m_per), :]
            else:
                stage[slot, :, pl.ds(cols, n_sub)] = y_f32
                src = stage.at[slot, :, pl.ds(cols, n_sub)]
                dst = out_ref.at[pl.ds(rows, m_per), pl.ds(cols, n_sub)]
            cp = pltpu.make_async_copy(src, dst, stage_sems.at[slot])
            cp.start()
            stage_copies[slot] = cp

        def target(c):
            return (my ^ RLIST[c]) if c < N_DEV - 1 else my

        def w_copy(s):
            off = target(s // NSUB) * n_per + (s % NSUB) * n_sub
            return pltpu.make_async_copy(
                w_ref.at[:, pl.ds(off, n_sub)],
                w_bufs.at[s % 2],
                w_sems.at[s % 2],
            )

        w_copy(0).start()

        def drain(c):
            src = my ^ RLIST[c]
            recv = pltpu.make_async_remote_copy(
                src_ref=y_bufs.at[0],
                dst_ref=recv_bufs.at[src],
                send_sem=send_sems.at[0, 0],
                recv_sem=recv_sems.at[src],
                device_id=(src,),
                device_id_type=pl.DeviceIdType.MESH,
            )
            recv.wait_recv()
            stage_out(c % 2, src * m_per, recv_bufs[src].astype(jnp.float32))

        send_rdmas = []
        for s in range(N_DEV * NSUB):
            c, h = s // NSUB, s % NSUB
            if s + 1 < N_DEV * NSUB:
                w_copy(s + 1).start()
            w_copy(s).wait()

            y = jnp.dot(x_ref[...], w_bufs[s % 2],
                        preferred_element_type=jnp.float32)
            y = y * jax.nn.sigmoid(y)

            if c == N_DEV - 1:
                stage_out(h, my * m_per, y, cols=h * n_sub)
            else:
                y_bufs[c, :, pl.ds(h * n_sub, n_sub)] = y.astype(jnp.bfloat16)
                rdma = pltpu.make_async_remote_copy(
                    src_ref=y_bufs.at[c, :, pl.ds(h * n_sub, n_sub)],
                    dst_ref=recv_bufs.at[my, :, pl.ds(h * n_sub, n_sub)],
                    send_sem=send_sems.at[c, h],
                    recv_sem=recv_sems.at[my],
                    device_id=(target(c),),
                    device_id_type=pl.DeviceIdType.MESH,
                )
                rdma.start()
                send_rdmas.append(rdma)

        for c in range(N_DEV - 1):
            drain(c)
        for cp in stage_copies:
            cp.wait()
        for r in send_rdmas:
            r.wait_send()

    return pl.pallas_call(
        body,
        out_shape=jax.ShapeDtypeStruct((m, n_per), jnp.float32),
        in_specs=[
            pl.BlockSpec(memory_space=pltpu.VMEM),
            pl.BlockSpec(memory_space=pltpu.MemorySpace.HBM),
        ],
        out_specs=pl.BlockSpec(memory_space=pltpu.MemorySpace.HBM),
        scratch_shapes=[
            pltpu.VMEM((2, k, n_per // NSUB), jnp.float32),
            pltpu.VMEM((N_DEV - 1, m_per, n_per), jnp.bfloat16),
            pltpu.VMEM((N_DEV, m_per, n_per), jnp.bfloat16),
            pltpu.VMEM((2, m_per, n_per), jnp.float32),
            pltpu.SemaphoreType.DMA((2,)),
            pltpu.SemaphoreType.DMA((N_DEV - 1, NSUB)),
            pltpu.SemaphoreType.DMA((N_DEV,)),
            pltpu.SemaphoreType.DMA((2,)),
        ],
        compiler_params=pltpu.CompilerParams(
            vmem_limit_bytes=64 * 1024 * 1024,
        ),
    )(x, w_mat)
